# Initial kernel scaffold; baseline (speedup 1.0000x reference)
import jax
import jax.numpy as jnp
from jax.experimental import pallas as pl


def kernel(x_inp_core, edge_index_core, edge_attr_core, x_upd_core, Z_core, Z_block, node2graph_core, W_embed, b_embed, W_edge, b_edge, W_msg, b_msg, W1, b1, W2, b2):
    raise NotImplementedError("write your pallas kernel here")


# speedup vs baseline: 1.3101x; 1.3101x over previous
import functools

import jax
import jax.numpy as jnp
from jax import lax
from jax.experimental import pallas as pl
from jax.experimental.pallas import tpu as pltpu
from jax.experimental.pallas import tpu_sc as plsc

V, E, G, D, DE, NL = 10000, 320000, 128, 128, 16, 4

NC, NS = 2, 16
NW = NC * NS
EPW = E // NW
SREG = 10496
CH = 128
KCAP = 320
VPAD = KCAP * NW


def _iota16():
    return lax.broadcasted_iota(jnp.int32, (16,), 0)


def _sc_prep_body(src_hbm, dst_hbm, eid_p, src_p, dst_p, cnt_out,
                  srcfull, dstfull, stag_e, stag_s, stag_d, cntbuf,
                  posr, offr):
    cid = lax.axis_index("c")
    sid = lax.axis_index("s")
    wid = cid * NS + sid
    base = wid * EPW
    iota = _iota16()

    base_al = pl.multiple_of(base, 8)
    pltpu.sync_copy(src_hbm.at[pl.ds(base_al, EPW)], srcfull)
    pltpu.sync_copy(dst_hbm.at[pl.ds(base_al, EPW)], dstfull)

    zvec = jnp.zeros((16,), jnp.int32)

    def zbody(i, _):
        stag_e[pl.ds(i * 16, 16)] = zvec
        stag_s[pl.ds(i * 16, 16)] = zvec
        stag_d[pl.ds(i * 16, 16)] = zvec
        return 0

    lax.fori_loop(0, SREG // 16, zbody, 0)

    for w in range(NW):
        posr[w] = jnp.int32(0)

    def p1(g, _):
        dv = dstfull[pl.ds(g * 16, 16)]
        for j in range(16):
            w = lax.bitwise_and(dv[j], 31)
            posr[w] = posr[w] + 1
        return 0

    lax.fori_loop(0, EPW // 16, p1, 0)

    acc = jnp.int32(0)
    for w in range(NW):
        c_w = posr[w]
        offr[w] = acc
        posr[w] = acc
        acc = acc + lax.bitwise_and(c_w + 7, jnp.int32(-8))

    def p2(g, _):
        dv = dstfull[pl.ds(g * 16, 16)]
        sv = srcfull[pl.ds(g * 16, 16)]
        for j in range(16):
            d = dv[j]
            w = lax.bitwise_and(d, 31)
            p = posr[w]
            pa = lax.bitwise_and(p, jnp.int32(-16))
            sel = iota == lax.bitwise_and(p, 15)
            stag_e[pl.ds(pa, 16)] = jnp.where(
                sel, jnp.full((16,), base + g * 16 + j, jnp.int32),
                stag_e[pl.ds(pa, 16)])
            stag_s[pl.ds(pa, 16)] = jnp.where(
                sel, jnp.full((16,), sv[j], jnp.int32),
                stag_s[pl.ds(pa, 16)])
            stag_d[pl.ds(pa, 16)] = jnp.where(
                sel, jnp.full((16,), d, jnp.int32), stag_d[pl.ds(pa, 16)])
            posr[w] = p + 1
        return 0

    lax.fori_loop(0, EPW // 16, p2, 0)

    reg = pl.multiple_of(wid * SREG, 8)
    pltpu.sync_copy(stag_e, eid_p.at[pl.ds(reg, SREG)])
    pltpu.sync_copy(stag_s, src_p.at[pl.ds(reg, SREG)])
    pltpu.sync_copy(stag_d, dst_p.at[pl.ds(reg, SREG)])

    for w in range(NW):
        cntbuf[pl.ds(w * 16, 16)] = jnp.full((16,), posr[w] - offr[w],
                                             jnp.int32)
    pltpu.sync_copy(cntbuf, cnt_out.at[pl.ds(pl.multiple_of(wid * NW * 16, 8), NW * 16)])


@functools.cache
def _get_sc_prep():
    mesh = plsc.VectorSubcoreMesh(core_axis_name="c", subcore_axis_name="s")
    return pl.kernel(
        _sc_prep_body,
        out_type=(
            jax.ShapeDtypeStruct((NW * SREG,), jnp.int32),
            jax.ShapeDtypeStruct((NW * SREG,), jnp.int32),
            jax.ShapeDtypeStruct((NW * SREG,), jnp.int32),
            jax.ShapeDtypeStruct((NW * NW * 16,), jnp.int32),
        ),
        mesh=mesh,
        scratch_types=[
            pltpu.VMEM((EPW,), jnp.int32),
            pltpu.VMEM((EPW,), jnp.int32),
            pltpu.VMEM((SREG,), jnp.int32),
            pltpu.VMEM((SREG,), jnp.int32),
            pltpu.VMEM((SREG,), jnp.int32),
            pltpu.VMEM((NW * 16,), jnp.int32),
            pltpu.SMEM((32,), jnp.int32),
            pltpu.SMEM((32,), jnp.int32),
        ],
    )


def _sc_layer_body(h_hbm, e_hbm, eid_p, src_p, dst_p, cnt_hbm, out_hbm,
                   cntv, eidv, srcv, dstv, rows, erows, aggl, idxv,
                   n_arr, off_arr, sem_g, sem_e):
    cid = lax.axis_index("c")
    sid = lax.axis_index("s")
    wid = cid * NS + sid
    iota = _iota16()

    zvec = jnp.zeros((16,), jnp.float32)

    def zbody(r, _):
        for j in range(D // 16):
            aggl[r, pl.ds(16 * j, 16)] = zvec
        return 0

    lax.fori_loop(0, KCAP, zbody, 0)

    pltpu.sync_copy(cnt_hbm, cntv)

    def ext(s, _):
        n_arr[s] = cntv[pl.ds((s * NW + wid) * 16, 16)][0]

        def inner(w, acc):
            c = cntv[pl.ds((s * NW + w) * 16, 16)][0]
            return acc + lax.bitwise_and(c + 7, jnp.int32(-8))

        off_arr[s] = lax.fori_loop(0, wid, inner, jnp.int32(0))
        return 0

    lax.fori_loop(0, NW, ext, 0)

    def sbody(s, _):
        n = n_arr[s]
        blk = s * SREG + off_arr[s]
        nch = (n + CH - 1) // CH

        def cbody(c, _):
            off = pl.multiple_of(blk + c * CH, 8)
            pltpu.sync_copy(eid_p.at[pl.ds(off, CH)], eidv)
            pltpu.sync_copy(src_p.at[pl.ds(off, CH)], srcv)
            pltpu.sync_copy(dst_p.at[pl.ds(off, CH)], dstv)
            cg = pltpu.async_copy(h_hbm.at[srcv], rows, sem_g)
            ce = pltpu.async_copy(e_hbm.at[eidv], erows, sem_e)
            cg.wait()
            ce.wait()
            cbase = c * CH

            def ubody(u, _):
                dv_u = dstv[pl.ds(u * 16, 16)]
                for j in range(16):
                    eidx = u * 16 + j
                    row = lax.shift_right_logical(dv_u[j], 5)
                    vf = jnp.full(
                        (16,),
                        jnp.where(cbase + eidx < n, jnp.float32(1),
                                  jnp.float32(0)))
                    for cc in range(D // 16):
                        sl = pl.ds(cc * 16, 16)
                        mval = jnp.maximum(
                            rows[eidx, sl] + erows[eidx, sl], 0.0) * vf
                        aggl[row, sl] = aggl[row, sl] + mval
                return 0

            lax.fori_loop(0, CH // 16, ubody, 0)
            return 0

        lax.fori_loop(0, nch, cbody, 0)
        return 0

    lax.fori_loop(0, NW, sbody, 0)

    for c in range(KCAP // 80):
        def ib(k, _, c=c):
            idxv[pl.ds(k * 16, 16)] = (c * 80 + k * 16 + iota) * 32 + wid
            return 0

        lax.fori_loop(0, 5, ib, 0)
        pltpu.sync_copy(aggl.at[pl.ds(c * 80, 80)], out_hbm.at[idxv])


@functools.cache
def _get_sc_layer():
    mesh = plsc.VectorSubcoreMesh(core_axis_name="c", subcore_axis_name="s")
    return pl.kernel(
        _sc_layer_body,
        out_type=jax.ShapeDtypeStruct((VPAD, D), jnp.float32),
        mesh=mesh,
        scratch_types=[
            pltpu.VMEM((NW * NW * 16,), jnp.int32),
            pltpu.VMEM((CH,), jnp.int32),
            pltpu.VMEM((CH,), jnp.int32),
            pltpu.VMEM((CH,), jnp.int32),
            pltpu.VMEM((CH, D), jnp.float32),
            pltpu.VMEM((CH, D), jnp.float32),
            pltpu.VMEM((KCAP, D), jnp.float32),
            pltpu.VMEM((80,), jnp.int32),
            pltpu.SMEM((32,), jnp.int32),
            pltpu.SMEM((32,), jnp.int32),
            pltpu.SemaphoreType.DMA,
            pltpu.SemaphoreType.DMA,
        ],
    )


def _sc_prep(src, dst):
    return _get_sc_prep()(src, dst)


def _sc_layer(h, e, eid_p, src_p, dst_p, cnt):
    return _get_sc_layer()(h, e, eid_p, src_p, dst_p, cnt)


def _embed_body(xu_ref, zc_ref, zb_ref, n2g_ref, we_ref, be_ref, h_ref):
    iota = lax.broadcasted_iota(jnp.int32, (V, G), 1)
    maskf = (n2g_ref[...] == iota).astype(jnp.float32)
    g1 = jnp.dot(maskf, zc_ref[...], preferred_element_type=jnp.float32,
                 precision=lax.Precision.HIGHEST)
    g2 = jnp.dot(maskf, zb_ref[...], preferred_element_type=jnp.float32,
                 precision=lax.Precision.HIGHEST)
    xc = jnp.concatenate([xu_ref[...], g1, g2], axis=-1)
    t = jnp.dot(xc, we_ref[...], preferred_element_type=jnp.float32)
    h_ref[...] = jnp.maximum(t + be_ref[...], 0.0)


_embed = pl.pallas_call(
    _embed_body,
    out_shape=jax.ShapeDtypeStruct((V, D), jnp.float32),
)

EB = 4000


def _edge_body(ea_ref, w_ref, b_ref, e_ref):
    e_ref[...] = jnp.maximum(
        jnp.dot(ea_ref[...], w_ref[...], preferred_element_type=jnp.float32)
        + b_ref[...], 0.0)


_edge = pl.pallas_call(
    _edge_body,
    grid=(E // EB,),
    in_specs=[pl.BlockSpec((EB, DE), lambda i: (i, 0)),
              pl.BlockSpec((DE, D), lambda i: (0, 0)),
              pl.BlockSpec((1, D), lambda i: (0, 0))],
    out_specs=pl.BlockSpec((EB, D), lambda i: (i, 0)),
    out_shape=jax.ShapeDtypeStruct((E, D), jnp.float32),
)


def _upd_body(h_ref, agg_ref, w_ref, b_ref, out_ref):
    xc = jnp.concatenate([h_ref[...], agg_ref[...][:V]], axis=-1)
    t = jnp.dot(xc, w_ref[...], preferred_element_type=jnp.float32)
    out_ref[...] = h_ref[...] + jnp.maximum(t + b_ref[...], 0.0)


_upd = pl.pallas_call(
    _upd_body,
    out_shape=jax.ShapeDtypeStruct((V, D), jnp.float32),
)


def _head_body(h_ref, xi_ref, n2g_ref, w1_ref, b1_ref, w2_ref, b2_ref, p_ref):
    xc = jnp.concatenate([h_ref[...], xi_ref[...]], axis=-1)
    hid = jnp.maximum(
        jnp.dot(xc, w1_ref[...], preferred_element_type=jnp.float32)
        + b1_ref[...], 0.0)
    logit = jnp.dot(hid, w2_ref[...],
                    preferred_element_type=jnp.float32) + b2_ref[...]
    iota = lax.broadcasted_iota(jnp.int32, (V, G), 1)
    maskb = n2g_ref[...] == iota
    maskf = maskb.astype(jnp.float32)
    neg = jnp.float32(-1e30)
    mx = jnp.max(jnp.where(maskb, logit, neg), axis=0, keepdims=True)
    mxn = jnp.sum(maskf * mx, axis=1, keepdims=True)
    ex = jnp.exp(logit - mxn)
    den = jnp.sum(maskf * ex, axis=0, keepdims=True)
    denn = jnp.sum(maskf * den, axis=1, keepdims=True)
    p_ref[...] = ex / denn


_head = pl.pallas_call(
    _head_body,
    out_shape=jax.ShapeDtypeStruct((V, 1), jnp.float32),
)


def kernel(x_inp_core, edge_index_core, edge_attr_core, x_upd_core, Z_core,
           Z_block, node2graph_core, W_embed, b_embed, W_edge, b_edge, W_msg,
           b_msg, W1, b1, W2, b2):
    n2g = node2graph_core.reshape(V, 1)
    h = _embed(x_upd_core, Z_core, Z_block, n2g, W_embed,
               b_embed.reshape(1, D))
    e = _edge(edge_attr_core, W_edge, b_edge.reshape(1, D))
    src = edge_index_core[0]
    dst = edge_index_core[1]
    eid_p, src_p, dst_p, cnt = _sc_prep(src, dst)
    for l in range(NL):
        agg_full = _sc_layer(h, e, eid_p, src_p, dst_p, cnt)
        h = _upd(h, agg_full, W_msg[l], b_msg[l].reshape(1, D))
    p = _head(h, x_inp_core, n2g, W1, b1.reshape(1, D), W2, b2.reshape(1, 1))
    return p.reshape(V)


# speedup vs baseline: 1.4632x; 1.1168x over previous
import functools

import jax
import jax.numpy as jnp
from jax import lax
from jax.experimental import pallas as pl
from jax.experimental.pallas import tpu as pltpu
from jax.experimental.pallas import tpu_sc as plsc

V, E, G, D, DE, NL = 10000, 320000, 128, 128, 16, 4

NC, NS = 2, 16
NW = NC * NS
EPW = E // NW
SREG = 10496
CH = 128
KCAP = 320
VPAD = KCAP * NW


def _iota16():
    return lax.broadcasted_iota(jnp.int32, (16,), 0)


def _sc_prep_body(src_hbm, dst_hbm, eid_p, src_p, dst_p, cnt_out,
                  srcfull, dstfull, stag_e, stag_s, stag_d, cntbuf,
                  posr, offr):
    cid = lax.axis_index("c")
    sid = lax.axis_index("s")
    wid = cid * NS + sid
    base = wid * EPW
    iota = _iota16()

    base_al = pl.multiple_of(base, 8)
    pltpu.sync_copy(src_hbm.at[pl.ds(base_al, EPW)], srcfull)
    pltpu.sync_copy(dst_hbm.at[pl.ds(base_al, EPW)], dstfull)

    zvec = jnp.zeros((16,), jnp.int32)

    def zbody(i, _):
        stag_e[pl.ds(i * 16, 16)] = zvec
        stag_s[pl.ds(i * 16, 16)] = zvec
        stag_d[pl.ds(i * 16, 16)] = zvec
        return 0

    lax.fori_loop(0, SREG // 16, zbody, 0)

    for w in range(NW):
        posr[w] = jnp.int32(0)

    def p1(g, _):
        dv = dstfull[pl.ds(g * 16, 16)]
        for j in range(16):
            w = lax.bitwise_and(dv[j], 31)
            posr[w] = posr[w] + 1
        return 0

    lax.fori_loop(0, EPW // 16, p1, 0)

    acc = jnp.int32(0)
    for w in range(NW):
        c_w = posr[w]
        offr[w] = acc
        posr[w] = acc
        acc = acc + lax.bitwise_and(c_w + 7, jnp.int32(-8))

    def p2(g, _):
        dv = dstfull[pl.ds(g * 16, 16)]
        sv = srcfull[pl.ds(g * 16, 16)]
        for j in range(16):
            d = dv[j]
            w = lax.bitwise_and(d, 31)
            p = posr[w]
            pa = lax.bitwise_and(p, jnp.int32(-16))
            sel = iota == lax.bitwise_and(p, 15)
            stag_e[pl.ds(pa, 16)] = jnp.where(
                sel, jnp.full((16,), base + g * 16 + j, jnp.int32),
                stag_e[pl.ds(pa, 16)])
            stag_s[pl.ds(pa, 16)] = jnp.where(
                sel, jnp.full((16,), sv[j], jnp.int32),
                stag_s[pl.ds(pa, 16)])
            stag_d[pl.ds(pa, 16)] = jnp.where(
                sel, jnp.full((16,), d, jnp.int32), stag_d[pl.ds(pa, 16)])
            posr[w] = p + 1
        return 0

    lax.fori_loop(0, EPW // 16, p2, 0)

    reg = pl.multiple_of(wid * SREG, 8)
    pltpu.sync_copy(stag_e, eid_p.at[pl.ds(reg, SREG)])
    pltpu.sync_copy(stag_s, src_p.at[pl.ds(reg, SREG)])
    pltpu.sync_copy(stag_d, dst_p.at[pl.ds(reg, SREG)])

    for w in range(NW):
        cntbuf[pl.ds(w * 16, 16)] = jnp.full((16,), posr[w] - offr[w],
                                             jnp.int32)
    pltpu.sync_copy(cntbuf, cnt_out.at[pl.ds(pl.multiple_of(wid * NW * 16, 8), NW * 16)])


@functools.cache
def _get_sc_prep():
    mesh = plsc.VectorSubcoreMesh(core_axis_name="c", subcore_axis_name="s")
    return pl.kernel(
        _sc_prep_body,
        out_type=(
            jax.ShapeDtypeStruct((NW * SREG,), jnp.int32),
            jax.ShapeDtypeStruct((NW * SREG,), jnp.int32),
            jax.ShapeDtypeStruct((NW * SREG,), jnp.int32),
            jax.ShapeDtypeStruct((NW * NW * 16,), jnp.int32),
        ),
        mesh=mesh,
        scratch_types=[
            pltpu.VMEM((EPW,), jnp.int32),
            pltpu.VMEM((EPW,), jnp.int32),
            pltpu.VMEM((SREG,), jnp.int32),
            pltpu.VMEM((SREG,), jnp.int32),
            pltpu.VMEM((SREG,), jnp.int32),
            pltpu.VMEM((NW * 16,), jnp.int32),
            pltpu.SMEM((32,), jnp.int32),
            pltpu.SMEM((32,), jnp.int32),
        ],
    )


def _sc_layer_body(h_hbm, e_hbm, eid_p, src_p, dst_p, cnt_hbm, out_hbm,
                   cntv, eidv, srcv, dstv, rows, erows, aggl, idxv,
                   n_arr, off_arr, sem_g, sem_e, sem_i):
    cid = lax.axis_index("c")
    sid = lax.axis_index("s")
    wid = cid * NS + sid
    iota = _iota16()

    zvec = jnp.zeros((16,), jnp.float32)

    def zbody(r, _):
        for j in range(D // 16):
            aggl[r, pl.ds(16 * j, 16)] = zvec
        return 0

    lax.fori_loop(0, KCAP, zbody, 0)

    pltpu.sync_copy(cnt_hbm, cntv)

    def ext(s, _):
        n_arr[s] = cntv[pl.ds((s * NW + wid) * 16, 16)][0]

        def inner(w, acc):
            c = cntv[pl.ds((s * NW + w) * 16, 16)][0]
            return acc + lax.bitwise_and(c + 7, jnp.int32(-8))

        off_arr[s] = lax.fori_loop(0, wid, inner, jnp.int32(0))
        return 0

    lax.fori_loop(0, NW, ext, 0)

    def sbody(s, _):
        n = n_arr[s]
        blk = s * SREG + off_arr[s]
        nch = (n + CH - 1) // CH

        def cbody(c, _):
            off = pl.multiple_of(blk + c * CH, 8)
            c1 = pltpu.async_copy(eid_p.at[pl.ds(off, CH)], eidv, sem_i)
            c2 = pltpu.async_copy(src_p.at[pl.ds(off, CH)], srcv, sem_i)
            c3 = pltpu.async_copy(dst_p.at[pl.ds(off, CH)], dstv, sem_i)
            c1.wait()
            c2.wait()
            c3.wait()
            cg = pltpu.async_copy(h_hbm.at[srcv], rows, sem_g)
            ce = pltpu.async_copy(e_hbm.at[eidv], erows, sem_e)
            cg.wait()
            ce.wait()
            cbase = c * CH

            def ubody_fast(u, _):
                dv_u = dstv[pl.ds(u * 16, 16)]
                for j in range(16):
                    eidx = u * 16 + j
                    row = lax.shift_right_logical(dv_u[j], 5)
                    for cc in range(D // 16):
                        sl = pl.ds(cc * 16, 16)
                        mval = jnp.maximum(
                            rows[eidx, sl] + erows[eidx, sl], 0.0)
                        aggl[row, sl] = aggl[row, sl] + mval
                return 0

            def ubody_masked(u, _):
                dv_u = dstv[pl.ds(u * 16, 16)]
                for j in range(16):
                    eidx = u * 16 + j
                    row = lax.shift_right_logical(dv_u[j], 5)
                    vf = jnp.full(
                        (16,),
                        jnp.where(cbase + eidx < n, jnp.float32(1),
                                  jnp.float32(0)))
                    for cc in range(D // 16):
                        sl = pl.ds(cc * 16, 16)
                        mval = jnp.maximum(
                            rows[eidx, sl] + erows[eidx, sl], 0.0) * vf
                        aggl[row, sl] = aggl[row, sl] + mval
                return 0

            whole = cbase + CH <= n

            @pl.when(whole)
            def _():
                lax.fori_loop(0, CH // 16, ubody_fast, 0)

            @pl.when(jnp.logical_not(whole))
            def _():
                lax.fori_loop(0, CH // 16, ubody_masked, 0)

            return 0

        lax.fori_loop(0, nch, cbody, 0)
        return 0

    lax.fori_loop(0, NW, sbody, 0)

    for c in range(KCAP // 80):
        def ib(k, _, c=c):
            idxv[pl.ds(k * 16, 16)] = (c * 80 + k * 16 + iota) * 32 + wid
            return 0

        lax.fori_loop(0, 5, ib, 0)
        pltpu.sync_copy(aggl.at[pl.ds(c * 80, 80)], out_hbm.at[idxv])


@functools.cache
def _get_sc_layer():
    mesh = plsc.VectorSubcoreMesh(core_axis_name="c", subcore_axis_name="s")
    return pl.kernel(
        _sc_layer_body,
        out_type=jax.ShapeDtypeStruct((VPAD, D), jnp.float32),
        mesh=mesh,
        scratch_types=[
            pltpu.VMEM((NW * NW * 16,), jnp.int32),
            pltpu.VMEM((CH,), jnp.int32),
            pltpu.VMEM((CH,), jnp.int32),
            pltpu.VMEM((CH,), jnp.int32),
            pltpu.VMEM((CH, D), jnp.float32),
            pltpu.VMEM((CH, D), jnp.float32),
            pltpu.VMEM((KCAP, D), jnp.float32),
            pltpu.VMEM((80,), jnp.int32),
            pltpu.SMEM((32,), jnp.int32),
            pltpu.SMEM((32,), jnp.int32),
            pltpu.SemaphoreType.DMA,
            pltpu.SemaphoreType.DMA,
            pltpu.SemaphoreType.DMA,
        ],
    )


def _sc_prep(src, dst):
    return _get_sc_prep()(src, dst)


def _sc_layer(h, e, eid_p, src_p, dst_p, cnt):
    return _get_sc_layer()(h, e, eid_p, src_p, dst_p, cnt)


def _embed_body(xu_ref, zc_ref, zb_ref, n2g_ref, we_ref, be_ref, h_ref):
    iota = lax.broadcasted_iota(jnp.int32, (V, G), 1)
    maskf = (n2g_ref[...] == iota).astype(jnp.float32)
    g1 = jnp.dot(maskf, zc_ref[...], preferred_element_type=jnp.float32,
                 precision=lax.Precision.HIGHEST)
    g2 = jnp.dot(maskf, zb_ref[...], preferred_element_type=jnp.float32,
                 precision=lax.Precision.HIGHEST)
    xc = jnp.concatenate([xu_ref[...], g1, g2], axis=-1)
    t = jnp.dot(xc, we_ref[...], preferred_element_type=jnp.float32)
    h_ref[...] = jnp.maximum(t + be_ref[...], 0.0)


_embed = pl.pallas_call(
    _embed_body,
    out_shape=jax.ShapeDtypeStruct((V, D), jnp.float32),
)

EB = 4000


def _edge_body(ea_ref, w_ref, b_ref, e_ref):
    e_ref[...] = jnp.maximum(
        jnp.dot(ea_ref[...], w_ref[...], preferred_element_type=jnp.float32)
        + b_ref[...], 0.0)


_edge = pl.pallas_call(
    _edge_body,
    grid=(E // EB,),
    in_specs=[pl.BlockSpec((EB, DE), lambda i: (i, 0)),
              pl.BlockSpec((DE, D), lambda i: (0, 0)),
              pl.BlockSpec((1, D), lambda i: (0, 0))],
    out_specs=pl.BlockSpec((EB, D), lambda i: (i, 0)),
    out_shape=jax.ShapeDtypeStruct((E, D), jnp.float32),
)


def _upd_body(h_ref, agg_ref, w_ref, b_ref, out_ref):
    xc = jnp.concatenate([h_ref[...], agg_ref[...][:V]], axis=-1)
    t = jnp.dot(xc, w_ref[...], preferred_element_type=jnp.float32)
    out_ref[...] = h_ref[...] + jnp.maximum(t + b_ref[...], 0.0)


_upd = pl.pallas_call(
    _upd_body,
    out_shape=jax.ShapeDtypeStruct((V, D), jnp.float32),
)


def _head_body(h_ref, xi_ref, n2g_ref, w1_ref, b1_ref, w2_ref, b2_ref, p_ref):
    xc = jnp.concatenate([h_ref[...], xi_ref[...]], axis=-1)
    hid = jnp.maximum(
        jnp.dot(xc, w1_ref[...], preferred_element_type=jnp.float32)
        + b1_ref[...], 0.0)
    logit = jnp.dot(hid, w2_ref[...],
                    preferred_element_type=jnp.float32) + b2_ref[...]
    iota = lax.broadcasted_iota(jnp.int32, (V, G), 1)
    maskb = n2g_ref[...] == iota
    maskf = maskb.astype(jnp.float32)
    neg = jnp.float32(-1e30)
    mx = jnp.max(jnp.where(maskb, logit, neg), axis=0, keepdims=True)
    mxn = jnp.sum(maskf * mx, axis=1, keepdims=True)
    ex = jnp.exp(logit - mxn)
    den = jnp.sum(maskf * ex, axis=0, keepdims=True)
    denn = jnp.sum(maskf * den, axis=1, keepdims=True)
    p_ref[...] = ex / denn


_head = pl.pallas_call(
    _head_body,
    out_shape=jax.ShapeDtypeStruct((V, 1), jnp.float32),
)


def kernel(x_inp_core, edge_index_core, edge_attr_core, x_upd_core, Z_core,
           Z_block, node2graph_core, W_embed, b_embed, W_edge, b_edge, W_msg,
           b_msg, W1, b1, W2, b2):
    n2g = node2graph_core.reshape(V, 1)
    h = _embed(x_upd_core, Z_core, Z_block, n2g, W_embed,
               b_embed.reshape(1, D))
    e = _edge(edge_attr_core, W_edge, b_edge.reshape(1, D))
    src = edge_index_core[0]
    dst = edge_index_core[1]
    eid_p, src_p, dst_p, cnt = _sc_prep(src, dst)
    for l in range(NL):
        agg_full = _sc_layer(h, e, eid_p, src_p, dst_p, cnt)
        h = _upd(h, agg_full, W_msg[l], b_msg[l].reshape(1, D))
    p = _head(h, x_inp_core, n2g, W1, b1.reshape(1, D), W2, b2.reshape(1, 1))
    return p.reshape(V)


# speedup vs baseline: 1.7838x; 1.2191x over previous
import functools

import jax
import jax.numpy as jnp
from jax import lax
from jax.experimental import pallas as pl
from jax.experimental.pallas import tpu as pltpu
from jax.experimental.pallas import tpu_sc as plsc

V, E, G, D, DE, NL = 10000, 320000, 128, 128, 16, 4

NC, NS = 2, 16
NW = NC * NS
EPW = E // NW
SREG = 10496
CH = 128
KCAP = 320
VPAD = KCAP * NW


def _iota16():
    return lax.broadcasted_iota(jnp.int32, (16,), 0)


def _sc_prep_body(src_hbm, dst_hbm, eid_p, src_p, dst_p, cnt_out,
                  srcfull, dstfull, stag_e, stag_s, stag_d, cntbuf,
                  posr, offr):
    cid = lax.axis_index("c")
    sid = lax.axis_index("s")
    wid = cid * NS + sid
    base = wid * EPW
    iota = _iota16()

    base_al = pl.multiple_of(base, 8)
    pltpu.sync_copy(src_hbm.at[pl.ds(base_al, EPW)], srcfull)
    pltpu.sync_copy(dst_hbm.at[pl.ds(base_al, EPW)], dstfull)

    zvec = jnp.zeros((16,), jnp.int32)

    def zbody(i, _):
        stag_e[pl.ds(i * 16, 16)] = zvec
        stag_s[pl.ds(i * 16, 16)] = zvec
        stag_d[pl.ds(i * 16, 16)] = zvec
        return 0

    lax.fori_loop(0, SREG // 16, zbody, 0)

    for w in range(NW):
        posr[w] = jnp.int32(0)

    def p1(g, _):
        dv = dstfull[pl.ds(g * 16, 16)]
        for j in range(16):
            w = lax.bitwise_and(dv[j], 31)
            posr[w] = posr[w] + 1
        return 0

    lax.fori_loop(0, EPW // 16, p1, 0)

    acc = jnp.int32(0)
    for w in range(NW):
        c_w = posr[w]
        offr[w] = acc
        posr[w] = acc
        acc = acc + lax.bitwise_and(c_w + 7, jnp.int32(-8))

    def p2(g, _):
        dv = dstfull[pl.ds(g * 16, 16)]
        sv = srcfull[pl.ds(g * 16, 16)]
        for j in range(16):
            d = dv[j]
            w = lax.bitwise_and(d, 31)
            p = posr[w]
            pa = lax.bitwise_and(p, jnp.int32(-16))
            sel = iota == lax.bitwise_and(p, 15)
            stag_e[pl.ds(pa, 16)] = jnp.where(
                sel, jnp.full((16,), base + g * 16 + j, jnp.int32),
                stag_e[pl.ds(pa, 16)])
            stag_s[pl.ds(pa, 16)] = jnp.where(
                sel, jnp.full((16,), sv[j], jnp.int32),
                stag_s[pl.ds(pa, 16)])
            stag_d[pl.ds(pa, 16)] = jnp.where(
                sel, jnp.full((16,), d, jnp.int32), stag_d[pl.ds(pa, 16)])
            posr[w] = p + 1
        return 0

    lax.fori_loop(0, EPW // 16, p2, 0)

    reg = pl.multiple_of(wid * SREG, 8)
    pltpu.sync_copy(stag_e, eid_p.at[pl.ds(reg, SREG)])
    pltpu.sync_copy(stag_s, src_p.at[pl.ds(reg, SREG)])
    pltpu.sync_copy(stag_d, dst_p.at[pl.ds(reg, SREG)])

    for w in range(NW):
        cntbuf[pl.ds(w * 16, 16)] = jnp.full((16,), posr[w] - offr[w],
                                             jnp.int32)
    pltpu.sync_copy(cntbuf, cnt_out.at[pl.ds(pl.multiple_of(wid * NW * 16, 8), NW * 16)])


@functools.cache
def _get_sc_prep():
    mesh = plsc.VectorSubcoreMesh(core_axis_name="c", subcore_axis_name="s")
    return pl.kernel(
        _sc_prep_body,
        out_type=(
            jax.ShapeDtypeStruct((NW * SREG,), jnp.int32),
            jax.ShapeDtypeStruct((NW * SREG,), jnp.int32),
            jax.ShapeDtypeStruct((NW * SREG,), jnp.int32),
            jax.ShapeDtypeStruct((NW * NW * 16,), jnp.int32),
        ),
        mesh=mesh,
        scratch_types=[
            pltpu.VMEM((EPW,), jnp.int32),
            pltpu.VMEM((EPW,), jnp.int32),
            pltpu.VMEM((SREG,), jnp.int32),
            pltpu.VMEM((SREG,), jnp.int32),
            pltpu.VMEM((SREG,), jnp.int32),
            pltpu.VMEM((NW * 16,), jnp.int32),
            pltpu.SMEM((32,), jnp.int32),
            pltpu.SMEM((32,), jnp.int32),
        ],
    )


def _sc_layer_body(h_hbm, e_hbm, eid_p, src_p, dst_p, cnt_hbm, out_hbm,
                   cntv, eidv, srcv, dstv, rows, erows, eidv2, srcv2, dstv2,
                   rows2, erows2, aggl, idxv, n_arr, off_arr, sem_g, sem_e,
                   sem_g2, sem_e2, sem_i):
    cid = lax.axis_index("c")
    sid = lax.axis_index("s")
    wid = cid * NS + sid
    iota = _iota16()
    bufs = ((eidv, srcv, dstv, rows, erows, sem_g, sem_e),
            (eidv2, srcv2, dstv2, rows2, erows2, sem_g2, sem_e2))

    zvec = jnp.zeros((16,), jnp.float32)

    def zbody(r, _):
        for j in range(D // 16):
            aggl[r, pl.ds(16 * j, 16)] = zvec
        return 0

    lax.fori_loop(0, KCAP, zbody, 0)

    pltpu.sync_copy(cnt_hbm, cntv)

    def ext(s, _):
        n_arr[s] = cntv[pl.ds((s * NW + wid) * 16, 16)][0]

        def inner(w, acc):
            c = cntv[pl.ds((s * NW + w) * 16, 16)][0]
            return acc + lax.bitwise_and(c + 7, jnp.int32(-8))

        off_arr[s] = lax.fori_loop(0, wid, inner, jnp.int32(0))
        return 0

    lax.fori_loop(0, NW, ext, 0)

    def sbody(s, _):
        n = n_arr[s]
        blk = s * SREG + off_arr[s]
        nch = (n + CH - 1) // CH

        def issue(c, b):
            ev, sv, dv, ro, er, sg, se = bufs[b]
            off = pl.multiple_of(blk + c * CH, 8)
            i1 = pltpu.async_copy(eid_p.at[pl.ds(off, CH)], ev, sem_i)
            i2 = pltpu.async_copy(src_p.at[pl.ds(off, CH)], sv, sem_i)
            i3 = pltpu.async_copy(dst_p.at[pl.ds(off, CH)], dv, sem_i)
            i1.wait()
            i2.wait()
            i3.wait()
            pltpu.async_copy(h_hbm.at[sv], ro, sg)
            pltpu.async_copy(e_hbm.at[ev], er, se)

        def compute(c, b):
            ev, sv, dv, ro, er, sg, se = bufs[b]
            pltpu.make_async_copy(h_hbm.at[sv], ro, sg).wait()
            pltpu.make_async_copy(e_hbm.at[ev], er, se).wait()
            cbase = c * CH

            def ubody_fast(u, _):
                dv_u = dv[pl.ds(u * 16, 16)]
                for j in range(16):
                    eidx = u * 16 + j
                    row = lax.shift_right_logical(dv_u[j], 5)
                    for cc in range(D // 16):
                        sl = pl.ds(cc * 16, 16)
                        mval = jnp.maximum(ro[eidx, sl] + er[eidx, sl], 0.0)
                        aggl[row, sl] = aggl[row, sl] + mval
                return 0

            def ubody_masked(u, _):
                dv_u = dv[pl.ds(u * 16, 16)]
                for j in range(16):
                    eidx = u * 16 + j
                    row = lax.shift_right_logical(dv_u[j], 5)
                    vf = jnp.full(
                        (16,),
                        jnp.where(cbase + eidx < n, jnp.float32(1),
                                  jnp.float32(0)))
                    for cc in range(D // 16):
                        sl = pl.ds(cc * 16, 16)
                        mval = jnp.maximum(
                            ro[eidx, sl] + er[eidx, sl], 0.0) * vf
                        aggl[row, sl] = aggl[row, sl] + mval
                return 0

            whole = cbase + CH <= n

            @pl.when(whole)
            def _():
                lax.fori_loop(0, CH // 16, ubody_fast, 0)

            @pl.when(jnp.logical_not(whole))
            def _():
                lax.fori_loop(0, CH // 16, ubody_masked, 0)

        @pl.when(nch > 0)
        def _():
            issue(0, 0)

            def cbody(c, _):
                b = lax.rem(c, 2)

                @pl.when(jnp.logical_and(c + 1 < nch, b == 0))
                def _():
                    issue(c + 1, 1)

                @pl.when(jnp.logical_and(c + 1 < nch, b == 1))
                def _():
                    issue(c + 1, 0)

                @pl.when(b == 0)
                def _():
                    compute(c, 0)

                @pl.when(b == 1)
                def _():
                    compute(c, 1)

                return 0

            lax.fori_loop(0, nch, cbody, 0)
        return 0

    lax.fori_loop(0, NW, sbody, 0)

    for c in range(KCAP // 80):
        def ib(k, _, c=c):
            idxv[pl.ds(k * 16, 16)] = (c * 80 + k * 16 + iota) * 32 + wid
            return 0

        lax.fori_loop(0, 5, ib, 0)
        pltpu.sync_copy(aggl.at[pl.ds(c * 80, 80)], out_hbm.at[idxv])


@functools.cache
def _get_sc_layer():
    mesh = plsc.VectorSubcoreMesh(core_axis_name="c", subcore_axis_name="s")
    return pl.kernel(
        _sc_layer_body,
        out_type=jax.ShapeDtypeStruct((VPAD, D), jnp.float32),
        mesh=mesh,
        scratch_types=[
            pltpu.VMEM((NW * NW * 16,), jnp.int32),
            pltpu.VMEM((CH,), jnp.int32),
            pltpu.VMEM((CH,), jnp.int32),
            pltpu.VMEM((CH,), jnp.int32),
            pltpu.VMEM((CH, D), jnp.float32),
            pltpu.VMEM((CH, D), jnp.float32),
            pltpu.VMEM((CH,), jnp.int32),
            pltpu.VMEM((CH,), jnp.int32),
            pltpu.VMEM((CH,), jnp.int32),
            pltpu.VMEM((CH, D), jnp.float32),
            pltpu.VMEM((CH, D), jnp.float32),
            pltpu.VMEM((KCAP, D), jnp.float32),
            pltpu.VMEM((80,), jnp.int32),
            pltpu.SMEM((32,), jnp.int32),
            pltpu.SMEM((32,), jnp.int32),
            pltpu.SemaphoreType.DMA,
            pltpu.SemaphoreType.DMA,
            pltpu.SemaphoreType.DMA,
            pltpu.SemaphoreType.DMA,
            pltpu.SemaphoreType.DMA,
        ],
    )


def _sc_prep(src, dst):
    return _get_sc_prep()(src, dst)


def _sc_layer(h, e, eid_p, src_p, dst_p, cnt):
    return _get_sc_layer()(h, e, eid_p, src_p, dst_p, cnt)


def _embed_body(xu_ref, zc_ref, zb_ref, n2g_ref, we_ref, be_ref, h_ref):
    iota = lax.broadcasted_iota(jnp.int32, (V, G), 1)
    maskf = (n2g_ref[...] == iota).astype(jnp.float32)
    g1 = jnp.dot(maskf, zc_ref[...], preferred_element_type=jnp.float32,
                 precision=lax.Precision.HIGHEST)
    g2 = jnp.dot(maskf, zb_ref[...], preferred_element_type=jnp.float32,
                 precision=lax.Precision.HIGHEST)
    xc = jnp.concatenate([xu_ref[...], g1, g2], axis=-1)
    t = jnp.dot(xc, we_ref[...], preferred_element_type=jnp.float32)
    h_ref[...] = jnp.maximum(t + be_ref[...], 0.0)


_embed = pl.pallas_call(
    _embed_body,
    out_shape=jax.ShapeDtypeStruct((V, D), jnp.float32),
)

EB = 4000


def _edge_body(ea_ref, w_ref, b_ref, e_ref):
    e_ref[...] = jnp.maximum(
        jnp.dot(ea_ref[...], w_ref[...], preferred_element_type=jnp.float32)
        + b_ref[...], 0.0)


_edge = pl.pallas_call(
    _edge_body,
    grid=(E // EB,),
    in_specs=[pl.BlockSpec((EB, DE), lambda i: (i, 0)),
              pl.BlockSpec((DE, D), lambda i: (0, 0)),
              pl.BlockSpec((1, D), lambda i: (0, 0))],
    out_specs=pl.BlockSpec((EB, D), lambda i: (i, 0)),
    out_shape=jax.ShapeDtypeStruct((E, D), jnp.float32),
)


def _upd_body(h_ref, agg_ref, w_ref, b_ref, out_ref):
    xc = jnp.concatenate([h_ref[...], agg_ref[...][:V]], axis=-1)
    t = jnp.dot(xc, w_ref[...], preferred_element_type=jnp.float32)
    out_ref[...] = h_ref[...] + jnp.maximum(t + b_ref[...], 0.0)


_upd = pl.pallas_call(
    _upd_body,
    out_shape=jax.ShapeDtypeStruct((V, D), jnp.float32),
)


def _head_body(h_ref, xi_ref, n2g_ref, w1_ref, b1_ref, w2_ref, b2_ref, p_ref):
    xc = jnp.concatenate([h_ref[...], xi_ref[...]], axis=-1)
    hid = jnp.maximum(
        jnp.dot(xc, w1_ref[...], preferred_element_type=jnp.float32)
        + b1_ref[...], 0.0)
    logit = jnp.dot(hid, w2_ref[...],
                    preferred_element_type=jnp.float32) + b2_ref[...]
    iota = lax.broadcasted_iota(jnp.int32, (V, G), 1)
    maskb = n2g_ref[...] == iota
    maskf = maskb.astype(jnp.float32)
    neg = jnp.float32(-1e30)
    mx = jnp.max(jnp.where(maskb, logit, neg), axis=0, keepdims=True)
    mxn = jnp.sum(maskf * mx, axis=1, keepdims=True)
    ex = jnp.exp(logit - mxn)
    den = jnp.sum(maskf * ex, axis=0, keepdims=True)
    denn = jnp.sum(maskf * den, axis=1, keepdims=True)
    p_ref[...] = ex / denn


_head = pl.pallas_call(
    _head_body,
    out_shape=jax.ShapeDtypeStruct((V, 1), jnp.float32),
)


def kernel(x_inp_core, edge_index_core, edge_attr_core, x_upd_core, Z_core,
           Z_block, node2graph_core, W_embed, b_embed, W_edge, b_edge, W_msg,
           b_msg, W1, b1, W2, b2):
    n2g = node2graph_core.reshape(V, 1)
    h = _embed(x_upd_core, Z_core, Z_block, n2g, W_embed,
               b_embed.reshape(1, D))
    e = _edge(edge_attr_core, W_edge, b_edge.reshape(1, D))
    src = edge_index_core[0]
    dst = edge_index_core[1]
    eid_p, src_p, dst_p, cnt = _sc_prep(src, dst)
    for l in range(NL):
        agg_full = _sc_layer(h, e, eid_p, src_p, dst_p, cnt)
        h = _upd(h, agg_full, W_msg[l], b_msg[l].reshape(1, D))
    p = _head(h, x_inp_core, n2g, W1, b1.reshape(1, D), W2, b2.reshape(1, 1))
    return p.reshape(V)
